# baseline (speedup 1.0000x reference)
import functools

import jax
import jax.numpy as jnp
from jax import lax
from jax.experimental import pallas as pl
from jax.experimental.pallas import tpu as pltpu
from jax.experimental.pallas import tpu_sc as plsc

D_NUM = 64
D_EMB = 128
D_OUT = D_NUM + 3 * D_EMB
NB = 2


@functools.lru_cache(maxsize=None)
def _make_kernel(B: int, L: int):
    info = plsc.get_sparse_core_info()
    NC, NS = info.num_cores, info.num_subcores
    NW = NC * NS
    assert B % (NW * NB * 2) == 0
    per_w = B // NW
    n_iter = per_w // NB

    mesh = plsc.VectorSubcoreMesh(core_axis_name="c", subcore_axis_name="s")

    slot_t = [
        pltpu.VMEM((NB, L), jnp.int32),
        pltpu.VMEM((NB, L), jnp.int32),
        pltpu.VMEM((NB, L), jnp.int32),
        pltpu.VMEM((NB, L, D_NUM), jnp.float32),
        pltpu.VMEM((NB, L, D_EMB), jnp.float32),
        pltpu.VMEM((NB, L, D_EMB), jnp.float32),
        pltpu.VMEM((NB, L, D_EMB), jnp.float32),
        pltpu.SemaphoreType.DMA,
        pltpu.SemaphoreType.DMA,
    ]

    @functools.partial(
        pl.kernel,
        mesh=mesh,
        out_type=jax.ShapeDtypeStruct((B, L, D_OUT), jnp.float32),
        scratch_types=slot_t + slot_t,
        compiler_params=pltpu.CompilerParams(use_tc_tiling_on_sc=False),
    )
    def k(numeric, wp_id, gl_id, ac_id, w_wp, w_gl, w_ac, out, *scratch):
        slots = (scratch[:9], scratch[9:])
        wid = lax.axis_index("s") * NC + lax.axis_index("c")
        w_base = wid * per_w

        def issue_in(s, b0):
            wi, gi, ai, num, wp, gl, ac, semg, _ = slots[s]
            pltpu.sync_copy(wp_id.at[pl.ds(b0, NB), :], wi)
            pltpu.sync_copy(gl_id.at[pl.ds(b0, NB), :], gi)
            pltpu.sync_copy(ac_id.at[pl.ds(b0, NB), :], ai)
            pltpu.async_copy(numeric.at[pl.ds(b0, NB), :, :], num, semg)
            for j in range(NB):
                pltpu.async_copy(w_wp.at[wi.at[j]], wp.at[j], semg)
                pltpu.async_copy(w_gl.at[gi.at[j]], gl.at[j], semg)
                pltpu.async_copy(w_ac.at[ai.at[j]], ac.at[j], semg)

        def wait_in(s, b0):
            wi, gi, ai, num, wp, gl, ac, semg, _ = slots[s]
            pltpu.make_async_copy(
                numeric.at[pl.ds(b0, NB), :, :], num, semg).wait()
            for j in range(NB):
                pltpu.make_async_copy(w_wp.at[wi.at[j]], wp.at[j], semg).wait()
                pltpu.make_async_copy(w_gl.at[gi.at[j]], gl.at[j], semg).wait()
                pltpu.make_async_copy(w_ac.at[ai.at[j]], ac.at[j], semg).wait()

        def out_copies(s, b0):
            _, _, _, num, wp, gl, ac, _, semw = slots[s]
            rows = pl.ds(b0, NB)
            return (
                pltpu.make_async_copy(
                    num, out.at[rows, :, pl.ds(0, D_NUM)], semw),
                pltpu.make_async_copy(
                    wp, out.at[rows, :, pl.ds(D_NUM, D_EMB)], semw),
                pltpu.make_async_copy(
                    gl, out.at[rows, :, pl.ds(D_NUM + D_EMB, D_EMB)], semw),
                pltpu.make_async_copy(
                    ac, out.at[rows, :, pl.ds(D_NUM + 2 * D_EMB, D_EMB)],
                    semw),
            )

        issue_in(0, w_base)
        issue_in(1, w_base + NB)

        def body(j, carry):
            for s in (0, 1):
                i = 2 * j + s
                b0 = w_base + i * NB
                wait_in(s, b0)
                copies = out_copies(s, b0)
                for c in copies:
                    c.start()
                for c in copies:
                    c.wait()

                @pl.when(i + 2 < n_iter)
                def _():
                    issue_in(s, b0 + 2 * NB)
            return carry

        lax.fori_loop(0, n_iter // 2, body, 0)

    return k


def kernel(numeric, waypoint_id, final_goal_id, action_id, W_wp, W_gl, W_ac):
    B, L, _ = numeric.shape
    wi = waypoint_id.astype(jnp.int32)
    gi = final_goal_id.astype(jnp.int32)
    ai = action_id.astype(jnp.int32)
    return _make_kernel(B, L)(numeric, wi, gi, ai, W_wp, W_gl, W_ac)


# speedup vs baseline: 1.0759x; 1.0759x over previous
import functools

import jax
import jax.numpy as jnp
from jax import lax
from jax.experimental import pallas as pl
from jax.experimental.pallas import tpu as pltpu
from jax.experimental.pallas import tpu_sc as plsc

D_NUM = 64
D_EMB = 128
D_OUT = D_NUM + 3 * D_EMB
NB = 2


@functools.lru_cache(maxsize=None)
def _make_kernel(B: int, L: int):
    info = plsc.get_sparse_core_info()
    NC, NS = info.num_cores, info.num_subcores
    NW = NC * NS
    assert B % (NW * NB) == 0
    per_w = B // NW
    n_iter = per_w // NB

    mesh = plsc.VectorSubcoreMesh(core_axis_name="c", subcore_axis_name="s")

    @functools.partial(
        pl.kernel,
        mesh=mesh,
        out_type=jax.ShapeDtypeStruct((B, L, D_OUT), jnp.float32),
        scratch_types=[
            pltpu.VMEM((NB, 128), jnp.int32),
            pltpu.VMEM((NB, 128), jnp.int32),
            pltpu.VMEM((NB, 128), jnp.int32),
            pltpu.VMEM((NB, L, D_EMB), jnp.float32),
            pltpu.VMEM((NB, L, D_EMB), jnp.float32),
            pltpu.VMEM((NB, L, D_EMB), jnp.float32),
            pltpu.VMEM((NB, L, D_EMB), jnp.float32),
            pltpu.VMEM((NB, L, D_EMB), jnp.float32),
            pltpu.VMEM((NB, L, D_EMB), jnp.float32),
            pltpu.VMEM((NB, L, D_NUM), jnp.float32),
            pltpu.SemaphoreType.DMA,
        ],
    )
    def k(numeric, wp_id, gl_id, ac_id, w_wp, w_gl, w_ac, out,
          wi, gi, ai, wpv, glv, acv, p0, p1, p2, p3, sem):
        wid = lax.axis_index("s") * NC + lax.axis_index("c")
        w_base = wid * per_w
        lidx = pl.ds(0, L)

        def body(i, carry):
            b0 = w_base + i * NB
            pltpu.sync_copy(wp_id.at[pl.ds(b0, NB), :], wi)
            pltpu.sync_copy(gl_id.at[pl.ds(b0, NB), :], gi)
            pltpu.sync_copy(ac_id.at[pl.ds(b0, NB), :], ai)
            cps = [pltpu.async_copy(numeric.at[pl.ds(b0, NB), :, :], p0, sem)]
            for j in range(NB):
                cps += [
                    pltpu.async_copy(w_wp.at[wi.at[j, lidx]], wpv.at[j], sem),
                    pltpu.async_copy(w_gl.at[gi.at[j, lidx]], glv.at[j], sem),
                    pltpu.async_copy(w_ac.at[ai.at[j, lidx]], acv.at[j], sem),
                ]
            for c in cps:
                c.wait()

            def shuffle(l, carry2):
                for j in range(NB):
                    for v in range(4):
                        s = pl.ds(16 * v, 16)
                        d = pl.ds(D_NUM + 16 * v, 16)
                        p0[j, l, d] = wpv[j, l, s]
                        p1[j, l, s] = wpv[j, l, d]
                        p1[j, l, d] = glv[j, l, s]
                        p2[j, l, s] = glv[j, l, d]
                        p2[j, l, d] = acv[j, l, s]
                        p3[j, l, s] = acv[j, l, d]
                return carry2

            lax.fori_loop(0, L, shuffle, 0)
            rows = pl.ds(b0, NB)
            pltpu.sync_copy(p0, out.at[rows, :, pl.ds(0, 128)])
            pltpu.sync_copy(p1, out.at[rows, :, pl.ds(128, 128)])
            pltpu.sync_copy(p2, out.at[rows, :, pl.ds(256, 128)])
            pltpu.sync_copy(p3, out.at[rows, :, pl.ds(384, 64)])
            return carry

        lax.fori_loop(0, n_iter, body, 0)

    return k


def kernel(numeric, waypoint_id, final_goal_id, action_id, W_wp, W_gl, W_ac):
    B, L, d_num = numeric.shape
    wi = jnp.pad(waypoint_id.astype(jnp.int32), ((0, 0), (0, 128 - L)))
    gi = jnp.pad(final_goal_id.astype(jnp.int32), ((0, 0), (0, 128 - L)))
    ai = jnp.pad(action_id.astype(jnp.int32), ((0, 0), (0, 128 - L)))
    num_p = jnp.pad(numeric, ((0, 0), (0, 0), (0, D_EMB - d_num)))
    return _make_kernel(B, L)(num_p, wi, gi, ai, W_wp, W_gl, W_ac)
